# baseline (speedup 1.0000x reference)
import jax
import jax.numpy as jnp
from jax import lax
from jax.experimental import pallas as pl
from jax.experimental.pallas import tpu as pltpu
from jax.experimental.pallas import tpu_sc as plsc

N = 4096
E = 4096
H = 128


_BM = 1024
_BN = 1024


def _a2_body(a_i, b_j, c_out, dtab_out, d2acc):
    i = pl.program_id(0)
    j = pl.program_id(1)
    nj = pl.num_programs(1)
    m11 = jnp.dot(a_i[:], b_j[:], preferred_element_type=jnp.float32)
    a1blk = a_i[:, pl.ds(j * _BN, _BN)].astype(jnp.float32)
    rows = i * _BM + lax.broadcasted_iota(jnp.int32, (_BM, _BN), 0)
    cols = j * _BN + lax.broadcasted_iota(jnp.int32, (_BM, _BN), 1)
    off_diag = rows != cols
    a2blk = jnp.where((m11 > 0.0) & (a1blk == 0.0) & off_diag, 1.0, 0.0)
    cblk = (a1blk + 2.0 * a2blk).astype(jnp.int32)
    c_out[:] = cblk[:, :_BN // 2] | lax.shift_left(cblk[:, _BN // 2:], 16)
    d2 = jnp.sum(a2blk, axis=1)

    @pl.when(j == 0)
    def _init():
        d2acc[:] = d2

    @pl.when(j > 0)
    def _acc():
        d2acc[:] += d2

    @pl.when(j == nj - 1)
    def _fin():
        deg1 = jnp.sum(a_i[:].astype(jnp.float32), axis=1)
        colid = lax.broadcasted_iota(jnp.int32, (_BM, 128), 1)
        dtab_out[:] = (jnp.where(colid == 0, deg1[:, None], 0.0)
                       + jnp.where(colid == 1, d2acc[:][:, None], 0.0))


def _compute_a2(a1_bf):
    grid = (N // _BM, N // _BN)
    return pl.pallas_call(
        _a2_body,
        grid=grid,
        in_specs=[
            pl.BlockSpec((_BM, N), lambda i, j: (i, 0)),
            pl.BlockSpec((N, _BN), lambda i, j: (0, j)),
        ],
        out_specs=[
            pl.BlockSpec((_BM, _BN // 2), lambda i, j: (i, j)),
            pl.BlockSpec((_BM, 128), lambda i, j: (i, 0)),
        ],
        out_shape=[
            jax.ShapeDtypeStruct((N, N // 2), jnp.int32),
            jax.ShapeDtypeStruct((N, 128), jnp.float32),
        ],
        scratch_shapes=[pltpu.VMEM((_BM,), jnp.float32)],
    )(a1_bf, a1_bf)


_NC = 2
_NS = 16
_NW = _NC * _NS
_EPW = E // _NW
_L = 16


def _sc_counts_body(u16_hbm, v16_hbm, c_hbm, dtab_hbm, out_hbm,
                    u16_v, v16_v,
                    cua, cva, dua, dva,
                    cub, cvb, dub, dvb,
                    col_v, sema, semb):
    wid = lax.axis_index("s") * _NC + lax.axis_index("c")
    base = wid * _EPW
    pltpu.sync_copy(u16_hbm.at[pl.ds(base * 16, _EPW * 16)], u16_v)
    pltpu.sync_copy(v16_hbm.at[pl.ds(base * 16, _EPW * 16)], v16_v)

    lane = lax.iota(jnp.int32, _L)
    lane_f = lane.astype(jnp.float32)
    zf = jnp.zeros((_L,), jnp.float32)
    zi = jnp.zeros((_L,), jnp.int32)
    oi = jnp.full((_L,), 1, jnp.int32)
    lomask = jnp.full((_L,), 65535, jnp.int32)
    bufs_a = (cua, cva, dua, dva)
    bufs_b = (cub, cvb, dub, dvb)

    def lsum(x):
        for s in (8, 4, 2, 1):
            x = x + x[lax.bitwise_and(lane + s, _L - 1)]
        return x

    def descriptors(e, bufs, sem):
        us = u16_v.at[pl.ds(e * 16, 1)]
        vs = v16_v.at[pl.ds(e * 16, 1)]
        return (pltpu.make_async_copy(c_hbm.at[us], bufs[0], sem),
                pltpu.make_async_copy(c_hbm.at[vs], bufs[1], sem),
                pltpu.make_async_copy(dtab_hbm.at[us], bufs[2], sem),
                pltpu.make_async_copy(dtab_hbm.at[vs], bufs[3], sem))

    def issue(e, bufs, sem):
        for d in descriptors(e, bufs, sem):
            d.start()

    def drain(e, bufs, sem):
        for d in descriptors(e, bufs, sem):
            d.wait()

    def compute(e, bufs):
        cu, cv, du, dv = bufs
        vv = v16_v[pl.ds(e * 16, _L)][zi]
        w_v = lax.shift_left(lax.shift_right_logical(vv, 10), 9) \
            + lax.bitwise_and(vv, 511)
        ish = lax.bitwise_and(lax.shift_right_logical(vv, 9), 1)
        nish = 1 - ish

        def chunk_body(c, accs):
            s1, s2, s3, s4, cuv, gv, gidx = accs
            for _ in range(8):
                sl = pl.ds(gidx, _L)
                wu = cu[0, sl]
                wv = cv[0, sl]
                plo = lax.bitwise_and(wu, lomask)
                phi = lax.shift_right_logical(wu, 16)
                qlo = lax.bitwise_and(wv, lomask)
                qhi = lax.shift_right_logical(wv, 16)
                pq = plo * qlo
                s1 = s1 + pq
                s2 = s2 + pq * plo
                s3 = s3 + pq * qlo
                s4 = s4 + pq * pq
                pq = phi * qhi
                s1 = s1 + pq
                s2 = s2 + pq * phi
                s3 = s3 + pq * qhi
                s4 = s4 + pq * pq
                pick = plo * nish + phi * ish
                cuv = cuv + jnp.where(gv == w_v, pick, 0)
                gv = gv + _L
                gidx = gidx + _L
            return (s1, s2, s3, s4, cuv, gv, gidx)

        accs = lax.fori_loop(0, (N // 2) // (8 * _L), chunk_body,
                             (zi,) * 5 + (lane, jnp.int32(0)))
        s1 = lsum(accs[0]).astype(jnp.float32)
        s2 = lsum(accs[1]).astype(jnp.float32)
        s3 = lsum(accs[2]).astype(jnp.float32)
        s4 = lsum(accs[3]).astype(jnp.float32)
        cuv = lsum(accs[4]).astype(jnp.float32)
        x22 = (s1 - s2 - s3 + s4) * 0.25
        x21 = (s2 - s1) * 0.5 - 2.0 * x22
        x12 = (s3 - s1) * 0.5 - 2.0 * x22
        x11 = s1 - 2.0 * x12 - 2.0 * x21 - 4.0 * x22
        a1uv = jnp.where(cuv == 1.0, 1.0, 0.0)
        a2uv = jnp.where(cuv == 2.0, 1.0, 0.0)
        duv = du[0, :]
        dvv = dv[0, :]
        y1u = duv[zi]
        y2u = duv[oi]
        y1v = dvv[zi]
        y2v = dvv[oi]
        c1i = y1u - x11 - x12 - a1uv
        ci1 = y1v - x11 - x21 - a1uv
        c2i = y2u - x21 - x22 - a2uv
        ci2 = y2v - x12 - x22 - a2uv
        tot = 2.0 + x11 + x12 + x21 + c1i + ci1 + x22 + c2i + ci2
        col = jnp.where(lane == 0, 2.0, 0.0)
        for k, ck in ((1, x11), (2, x12), (3, x21), (4, c1i), (5, ci1),
                      (6, x22), (7, c2i), (8, ci2), (9, tot)):
            col = jnp.where(lane == k, ck, col)
        col_v[pl.ds(e * 16, _L)] = col

    issue(0, bufs_a, sema)

    def pair_body(g, carry):
        e0 = g * 2
        drain(e0, bufs_a, sema)
        issue(e0 + 1, bufs_b, semb)
        compute(e0, bufs_a)
        drain(e0 + 1, bufs_b, semb)

        @pl.when(g < _EPW // 2 - 1)
        def _next():
            issue(e0 + 2, bufs_a, sema)

        compute(e0 + 1, bufs_b)
        return 0

    lax.fori_loop(0, _EPW // 2, pair_body, 0)
    pltpu.sync_copy(col_v, out_hbm.at[pl.ds(base * 16, _EPW * 16)])


def _sc_counts(u, v, cmat, dtab):
    u16 = jnp.zeros((E, 16), jnp.int32).at[:, 0].set(u).reshape(-1)
    v16 = jnp.zeros((E, 16), jnp.int32).at[:, 0].set(v).reshape(-1)
    mesh = plsc.VectorSubcoreMesh(core_axis_name="c", subcore_axis_name="s")
    f32 = jnp.float32
    rowbuf = pltpu.VMEM((1, N // 2), jnp.int32)
    degbuf = pltpu.VMEM((1, 128), f32)
    scratch = [
        pltpu.VMEM((_EPW * 16,), jnp.int32),
        pltpu.VMEM((_EPW * 16,), jnp.int32),
        rowbuf, rowbuf, degbuf, degbuf,
        rowbuf, rowbuf, degbuf, degbuf,
        pltpu.VMEM((_EPW * 16,), f32),
        pltpu.SemaphoreType.DMA,
        pltpu.SemaphoreType.DMA,
    ]
    fn = pl.kernel(
        _sc_counts_body,
        out_type=jax.ShapeDtypeStruct((E * 16,), f32),
        mesh=mesh,
        scratch_types=scratch,
    )
    return fn(u16, v16, cmat, dtab).reshape(E, 16)


_BE = 1024

_PAIRS = ((0, 0), (1, 1), (1, 2), (2, 1), (1, 3), (3, 1), (2, 2), (2, 3),
          (3, 2))


def _mlp_body(cnt, z, w1, b1, w2, b2, w3, b3, out):
    zt = z[:]
    t = jnp.stack([zt[a] + zt[b] for a, b in _PAIRS]
                  + [jnp.zeros((H,), jnp.float32)] * 7)
    cblk = cnt[:]
    pooled = jnp.dot(cblk, t, preferred_element_type=jnp.float32,
                     precision=lax.Precision.HIGHEST)
    total = cblk[:, 9:10]
    out_t = pooled / total
    h1 = jnp.maximum(jnp.dot(out_t, w1[:],
                             preferred_element_type=jnp.float32,
                             precision=lax.Precision.HIGHEST)
                     + b1[:][None, :], 0.0)
    h2 = jnp.maximum(jnp.dot(h1, w2[:],
                             preferred_element_type=jnp.float32,
                             precision=lax.Precision.HIGHEST)
                     + b2[:][None, :], 0.0)
    logit = jnp.dot(h2, w3[:],
                    preferred_element_type=jnp.float32,
                    precision=lax.Precision.HIGHEST) + b3[0]
    out[:] = jnp.broadcast_to(logit, (_BE, 128))


def _mlp_head(counts, z_table, W1, b1, W2, b2, W3, b3):
    grid = (E // _BE,)
    logit = pl.pallas_call(
        _mlp_body,
        grid=grid,
        in_specs=[
            pl.BlockSpec((_BE, 16), lambda e: (e, 0)),
            pl.BlockSpec((4, H), lambda e: (0, 0)),
            pl.BlockSpec((H, H), lambda e: (0, 0)),
            pl.BlockSpec((H,), lambda e: (0,)),
            pl.BlockSpec((H, H), lambda e: (0, 0)),
            pl.BlockSpec((H,), lambda e: (0,)),
            pl.BlockSpec((H, 1), lambda e: (0, 0)),
            pl.BlockSpec(memory_space=pltpu.SMEM),
        ],
        out_specs=pl.BlockSpec((_BE, 128), lambda e: (e, 0)),
        out_shape=jax.ShapeDtypeStruct((E, 128), jnp.float32),
    )(counts, z_table, W1, b1, W2, b2, W3, b3)
    return logit[:, :1]


def kernel(x, adj, edges, z_table, W1, b1, W2, b2, W3, b3):
    del x
    a1_bf = adj.astype(jnp.bfloat16)
    cmat, dtab = _compute_a2(a1_bf)
    u = edges[0]
    v = edges[1]
    counts = _sc_counts(u, v, cmat, dtab)
    return _mlp_head(counts, z_table, W1, b1, W2, b2, W3, b3)


# speedup vs baseline: 1.0321x; 1.0321x over previous
import jax
import jax.numpy as jnp
from jax import lax
from jax.experimental import pallas as pl
from jax.experimental.pallas import tpu as pltpu
from jax.experimental.pallas import tpu_sc as plsc

N = 4096
E = 4096
H = 128


_BM = 1024
_BN = 1024


def _a2_body(a_i, b_j, c_out, dtab_out, d2acc):
    i = pl.program_id(0)
    j = pl.program_id(1)
    nj = pl.num_programs(1)
    m11 = jnp.dot(a_i[:], b_j[:], preferred_element_type=jnp.float32)
    a1blk = a_i[:, pl.ds(j * _BN, _BN)].astype(jnp.float32)
    rows = i * _BM + lax.broadcasted_iota(jnp.int32, (_BM, _BN), 0)
    cols = j * _BN + lax.broadcasted_iota(jnp.int32, (_BM, _BN), 1)
    off_diag = rows != cols
    a2blk = jnp.where((m11 > 0.0) & (a1blk == 0.0) & off_diag, 1.0, 0.0)
    cblk = (a1blk + 2.0 * a2blk).astype(jnp.int32)
    c_out[:] = cblk[:, :_BN // 2] | lax.shift_left(cblk[:, _BN // 2:], 16)
    d2 = jnp.sum(a2blk, axis=1)

    @pl.when(j == 0)
    def _init():
        d2acc[:] = d2

    @pl.when(j > 0)
    def _acc():
        d2acc[:] += d2

    @pl.when(j == nj - 1)
    def _fin():
        deg1 = jnp.sum(a_i[:].astype(jnp.float32), axis=1)
        colid = lax.broadcasted_iota(jnp.int32, (_BM, 128), 1)
        dtab_out[:] = (jnp.where(colid == 0, deg1[:, None], 0.0)
                       + jnp.where(colid == 1, d2acc[:][:, None], 0.0))


def _compute_a2(a1_bf):
    grid = (N // _BM, N // _BN)
    return pl.pallas_call(
        _a2_body,
        grid=grid,
        in_specs=[
            pl.BlockSpec((_BM, N), lambda i, j: (i, 0)),
            pl.BlockSpec((N, _BN), lambda i, j: (0, j)),
        ],
        out_specs=[
            pl.BlockSpec((_BM, _BN // 2), lambda i, j: (i, j)),
            pl.BlockSpec((_BM, 128), lambda i, j: (i, 0)),
        ],
        out_shape=[
            jax.ShapeDtypeStruct((N, N // 2), jnp.int32),
            jax.ShapeDtypeStruct((N, 128), jnp.float32),
        ],
        scratch_shapes=[pltpu.VMEM((_BM,), jnp.float32)],
    )(a1_bf, a1_bf)


_NC = 2
_NS = 16
_NW = _NC * _NS
_EPW = E // _NW
_L = 16


def _sc_counts_body(u16_hbm, v16_hbm, c_hbm, dtab_hbm, out_hbm,
                    u16_v, v16_v,
                    cua, cva, dua, dva,
                    cub, cvb, dub, dvb,
                    col_v, sema, semb):
    wid = lax.axis_index("s") * _NC + lax.axis_index("c")
    base = wid * _EPW
    pltpu.sync_copy(u16_hbm.at[pl.ds(base * 16, _EPW * 16)], u16_v)
    pltpu.sync_copy(v16_hbm.at[pl.ds(base * 16, _EPW * 16)], v16_v)

    lane = lax.iota(jnp.int32, _L)
    lane_f = lane.astype(jnp.float32)
    zf = jnp.zeros((_L,), jnp.float32)
    zi = jnp.zeros((_L,), jnp.int32)
    oi = jnp.full((_L,), 1, jnp.int32)
    lomask = jnp.full((_L,), 65535, jnp.int32)
    bufs_a = (cua, cva, dua, dva)
    bufs_b = (cub, cvb, dub, dvb)

    def lsum(x):
        for s in (8, 4, 2, 1):
            x = x + x[lax.bitwise_and(lane + s, _L - 1)]
        return x

    def descriptors(e, bufs, sem):
        us = u16_v.at[pl.ds(e * 16, 1)]
        vs = v16_v.at[pl.ds(e * 16, 1)]
        return (pltpu.make_async_copy(c_hbm.at[us], bufs[0], sem),
                pltpu.make_async_copy(c_hbm.at[vs], bufs[1], sem),
                pltpu.make_async_copy(dtab_hbm.at[us], bufs[2], sem),
                pltpu.make_async_copy(dtab_hbm.at[vs], bufs[3], sem))

    def issue(e, bufs, sem):
        for d in descriptors(e, bufs, sem):
            d.start()

    def drain(e, bufs, sem):
        for d in descriptors(e, bufs, sem):
            d.wait()

    def compute(e, bufs):
        cu, cv, du, dv = bufs
        vv = v16_v[pl.ds(e * 16, _L)][zi]
        w_v = lax.shift_left(lax.shift_right_logical(vv, 10), 9) \
            + lax.bitwise_and(vv, 511)
        ish = lax.bitwise_and(lax.shift_right_logical(vv, 9), 1)
        nish = 1 - ish

        def chunk_body(c, accs):
            s1, s2, s3, s4, cuv, gv, gidx = accs
            for _ in range(4):
                sl = pl.ds(gidx, _L)
                wu = cu[0, sl]
                wv = cv[0, sl]
                plo = lax.bitwise_and(wu, lomask)
                phi = lax.shift_right_logical(wu, 16)
                qlo = lax.bitwise_and(wv, lomask)
                qhi = lax.shift_right_logical(wv, 16)
                pq = plo * qlo
                s1 = s1 + pq
                s2 = s2 + pq * plo
                s3 = s3 + pq * qlo
                s4 = s4 + pq * pq
                pq = phi * qhi
                s1 = s1 + pq
                s2 = s2 + pq * phi
                s3 = s3 + pq * qhi
                s4 = s4 + pq * pq
                pick = plo * nish + phi * ish
                cuv = cuv + jnp.where(gv == w_v, pick, 0)
                gv = gv + _L
                gidx = gidx + _L
            return (s1, s2, s3, s4, cuv, gv, gidx)

        accs = lax.fori_loop(0, (N // 2) // (4 * _L), chunk_body,
                             (zi,) * 5 + (lane, jnp.int32(0)))
        s1 = lsum(accs[0]).astype(jnp.float32)
        s2 = lsum(accs[1]).astype(jnp.float32)
        s3 = lsum(accs[2]).astype(jnp.float32)
        s4 = lsum(accs[3]).astype(jnp.float32)
        cuv = lsum(accs[4]).astype(jnp.float32)
        x22 = (s1 - s2 - s3 + s4) * 0.25
        x21 = (s2 - s1) * 0.5 - 2.0 * x22
        x12 = (s3 - s1) * 0.5 - 2.0 * x22
        x11 = s1 - 2.0 * x12 - 2.0 * x21 - 4.0 * x22
        a1uv = jnp.where(cuv == 1.0, 1.0, 0.0)
        a2uv = jnp.where(cuv == 2.0, 1.0, 0.0)
        duv = du[0, :]
        dvv = dv[0, :]
        y1u = duv[zi]
        y2u = duv[oi]
        y1v = dvv[zi]
        y2v = dvv[oi]
        c1i = y1u - x11 - x12 - a1uv
        ci1 = y1v - x11 - x21 - a1uv
        c2i = y2u - x21 - x22 - a2uv
        ci2 = y2v - x12 - x22 - a2uv
        tot = 2.0 + x11 + x12 + x21 + c1i + ci1 + x22 + c2i + ci2
        col = jnp.where(lane == 0, 2.0, 0.0)
        for k, ck in ((1, x11), (2, x12), (3, x21), (4, c1i), (5, ci1),
                      (6, x22), (7, c2i), (8, ci2), (9, tot)):
            col = jnp.where(lane == k, ck, col)
        col_v[pl.ds(e * 16, _L)] = col

    issue(0, bufs_a, sema)

    def pair_body(g, carry):
        e0 = g * 2
        drain(e0, bufs_a, sema)
        issue(e0 + 1, bufs_b, semb)
        compute(e0, bufs_a)
        drain(e0 + 1, bufs_b, semb)

        @pl.when(g < _EPW // 2 - 1)
        def _next():
            issue(e0 + 2, bufs_a, sema)

        compute(e0 + 1, bufs_b)
        return 0

    lax.fori_loop(0, _EPW // 2, pair_body, 0)
    pltpu.sync_copy(col_v, out_hbm.at[pl.ds(base * 16, _EPW * 16)])


def _sc_counts(u, v, cmat, dtab):
    u16 = jnp.zeros((E, 16), jnp.int32).at[:, 0].set(u).reshape(-1)
    v16 = jnp.zeros((E, 16), jnp.int32).at[:, 0].set(v).reshape(-1)
    mesh = plsc.VectorSubcoreMesh(core_axis_name="c", subcore_axis_name="s")
    f32 = jnp.float32
    rowbuf = pltpu.VMEM((1, N // 2), jnp.int32)
    degbuf = pltpu.VMEM((1, 128), f32)
    scratch = [
        pltpu.VMEM((_EPW * 16,), jnp.int32),
        pltpu.VMEM((_EPW * 16,), jnp.int32),
        rowbuf, rowbuf, degbuf, degbuf,
        rowbuf, rowbuf, degbuf, degbuf,
        pltpu.VMEM((_EPW * 16,), f32),
        pltpu.SemaphoreType.DMA,
        pltpu.SemaphoreType.DMA,
    ]
    fn = pl.kernel(
        _sc_counts_body,
        out_type=jax.ShapeDtypeStruct((E * 16,), f32),
        mesh=mesh,
        scratch_types=scratch,
    )
    return fn(u16, v16, cmat, dtab).reshape(E, 16)


_BE = 1024

_PAIRS = ((0, 0), (1, 1), (1, 2), (2, 1), (1, 3), (3, 1), (2, 2), (2, 3),
          (3, 2))


def _mlp_body(cnt, z, w1, b1, w2, b2, w3, b3, out):
    zt = z[:]
    t = jnp.stack([zt[a] + zt[b] for a, b in _PAIRS]
                  + [jnp.zeros((H,), jnp.float32)] * 7)
    cblk = cnt[:]
    pooled = jnp.dot(cblk, t, preferred_element_type=jnp.float32,
                     precision=lax.Precision.HIGHEST)
    total = cblk[:, 9:10]
    out_t = pooled / total
    h1 = jnp.maximum(jnp.dot(out_t, w1[:],
                             preferred_element_type=jnp.float32)
                     + b1[:][None, :], 0.0)
    h2 = jnp.maximum(jnp.dot(h1, w2[:],
                             preferred_element_type=jnp.float32)
                     + b2[:][None, :], 0.0)
    logit = jnp.dot(h2, w3[:],
                    preferred_element_type=jnp.float32) + b3[0]
    out[:] = jnp.broadcast_to(logit, (_BE, 128))


def _mlp_head(counts, z_table, W1, b1, W2, b2, W3, b3):
    grid = (E // _BE,)
    logit = pl.pallas_call(
        _mlp_body,
        grid=grid,
        in_specs=[
            pl.BlockSpec((_BE, 16), lambda e: (e, 0)),
            pl.BlockSpec((4, H), lambda e: (0, 0)),
            pl.BlockSpec((H, H), lambda e: (0, 0)),
            pl.BlockSpec((H,), lambda e: (0,)),
            pl.BlockSpec((H, H), lambda e: (0, 0)),
            pl.BlockSpec((H,), lambda e: (0,)),
            pl.BlockSpec((H, 1), lambda e: (0, 0)),
            pl.BlockSpec(memory_space=pltpu.SMEM),
        ],
        out_specs=pl.BlockSpec((_BE, 128), lambda e: (e, 0)),
        out_shape=jax.ShapeDtypeStruct((E, 128), jnp.float32),
    )(counts, z_table, W1, b1, W2, b2, W3, b3)
    return logit[:, :1]


def kernel(x, adj, edges, z_table, W1, b1, W2, b2, W3, b3):
    del x
    a1_bf = adj.astype(jnp.bfloat16)
    cmat, dtab = _compute_a2(a1_bf)
    u = edges[0]
    v = edges[1]
    counts = _sc_counts(u, v, cmat, dtab)
    return _mlp_head(counts, z_table, W1, b1, W2, b2, W3, b3)


# speedup vs baseline: 1.0370x; 1.0047x over previous
import jax
import jax.numpy as jnp
from jax import lax
from jax.experimental import pallas as pl
from jax.experimental.pallas import tpu as pltpu
from jax.experimental.pallas import tpu_sc as plsc

N = 4096
E = 4096
H = 128


_BM = 1024
_BN = 1024


def _a2_body(a_i, b_j, c_out, dtab_out, d2acc):
    i = pl.program_id(0)
    j = pl.program_id(1)
    nj = pl.num_programs(1)
    m11 = jnp.dot(a_i[:], b_j[:], preferred_element_type=jnp.float32)
    a1blk = a_i[:, pl.ds(j * _BN, _BN)].astype(jnp.float32)
    rows = i * _BM + lax.broadcasted_iota(jnp.int32, (_BM, _BN), 0)
    cols = j * _BN + lax.broadcasted_iota(jnp.int32, (_BM, _BN), 1)
    off_diag = rows != cols
    a2blk = jnp.where((m11 > 0.0) & (a1blk == 0.0) & off_diag, 1.0, 0.0)
    cblk = (a1blk + 2.0 * a2blk).astype(jnp.int32)
    c_out[:] = cblk[:, :_BN // 2] | lax.shift_left(cblk[:, _BN // 2:], 16)
    d2 = jnp.sum(a2blk, axis=1)

    @pl.when(j == 0)
    def _init():
        d2acc[:] = d2

    @pl.when(j > 0)
    def _acc():
        d2acc[:] += d2

    @pl.when(j == nj - 1)
    def _fin():
        deg1 = jnp.sum(a_i[:].astype(jnp.float32), axis=1)
        colid = lax.broadcasted_iota(jnp.int32, (_BM, 128), 1)
        dtab_out[:] = (jnp.where(colid == 0, deg1[:, None], 0.0)
                       + jnp.where(colid == 1, d2acc[:][:, None], 0.0))


def _compute_a2(a1_bf):
    grid = (N // _BM, N // _BN)
    return pl.pallas_call(
        _a2_body,
        grid=grid,
        in_specs=[
            pl.BlockSpec((_BM, N), lambda i, j: (i, 0)),
            pl.BlockSpec((N, _BN), lambda i, j: (0, j)),
        ],
        out_specs=[
            pl.BlockSpec((_BM, _BN // 2), lambda i, j: (i, j)),
            pl.BlockSpec((_BM, 128), lambda i, j: (i, 0)),
        ],
        out_shape=[
            jax.ShapeDtypeStruct((N, N // 2), jnp.int32),
            jax.ShapeDtypeStruct((N, 128), jnp.float32),
        ],
        scratch_shapes=[pltpu.VMEM((_BM,), jnp.float32)],
    )(a1_bf, a1_bf)


_NC = 2
_NS = 16
_NW = _NC * _NS
_EPW = E // _NW
_L = 16


def _sc_counts_body(u16_hbm, v16_hbm, c_hbm, dtab_hbm, out_hbm,
                    u16_v, v16_v,
                    cua, cva, dua, dva,
                    cub, cvb, dub, dvb,
                    col_v, sema, semb):
    wid = lax.axis_index("s") * _NC + lax.axis_index("c")
    base = wid * _EPW
    pltpu.sync_copy(u16_hbm.at[pl.ds(base * 16, _EPW * 16)], u16_v)
    pltpu.sync_copy(v16_hbm.at[pl.ds(base * 16, _EPW * 16)], v16_v)

    lane = lax.iota(jnp.int32, _L)
    lane_f = lane.astype(jnp.float32)
    zf = jnp.zeros((_L,), jnp.float32)
    zi = jnp.zeros((_L,), jnp.int32)
    oi = jnp.full((_L,), 1, jnp.int32)
    lomask = jnp.full((_L,), 65535, jnp.int32)
    bufs_a = (cua, cva, dua, dva)
    bufs_b = (cub, cvb, dub, dvb)

    def lsum(x):
        for s in (8, 4, 2, 1):
            x = x + x[lax.bitwise_and(lane + s, _L - 1)]
        return x

    def descriptors(e, bufs, sem):
        us = u16_v.at[pl.ds(e * 16, 1)]
        vs = v16_v.at[pl.ds(e * 16, 1)]
        return (pltpu.make_async_copy(c_hbm.at[us], bufs[0], sem),
                pltpu.make_async_copy(c_hbm.at[vs], bufs[1], sem),
                pltpu.make_async_copy(dtab_hbm.at[us], bufs[2], sem),
                pltpu.make_async_copy(dtab_hbm.at[vs], bufs[3], sem))

    def issue(e, bufs, sem):
        for d in descriptors(e, bufs, sem):
            d.start()

    def drain(e, bufs, sem):
        for d in descriptors(e, bufs, sem):
            d.wait()

    def compute(e, bufs):
        cu, cv, du, dv = bufs
        vv = v16_v[pl.ds(e * 16, _L)][zi]
        w_v = lax.shift_left(lax.shift_right_logical(vv, 10), 9) \
            + lax.bitwise_and(vv, 511)
        ish = lax.bitwise_and(lax.shift_right_logical(vv, 9), 1)
        nish = 1 - ish

        def chunk_body(c, accs):
            s1, s2, s3, s4, cuv, gv, gidx = accs
            for _ in range(4):
                sl = pl.ds(gidx, _L)
                wu = cu[0, sl]
                wv = cv[0, sl]
                plo = lax.bitwise_and(wu, lomask)
                phi = lax.shift_right_logical(wu, 16)
                qlo = lax.bitwise_and(wv, lomask)
                qhi = lax.shift_right_logical(wv, 16)
                pq = plo * qlo
                s1 = s1 + pq
                s2 = s2 + pq * plo
                s3 = s3 + pq * qlo
                s4 = s4 + pq * pq
                pq = phi * qhi
                s1 = s1 + pq
                s2 = s2 + pq * phi
                s3 = s3 + pq * qhi
                s4 = s4 + pq * pq
                pick = plo * nish + phi * ish
                cuv = cuv + jnp.where(gv == w_v, pick, 0)
                gv = gv + _L
                gidx = gidx + _L
            return (s1, s2, s3, s4, cuv, gv, gidx)

        accs = lax.fori_loop(0, (N // 2) // (4 * _L), chunk_body,
                             (zi,) * 5 + (lane, jnp.int32(0)))
        s1 = lsum(accs[0]).astype(jnp.float32)
        s2 = lsum(accs[1]).astype(jnp.float32)
        s3 = lsum(accs[2]).astype(jnp.float32)
        s4 = lsum(accs[3]).astype(jnp.float32)
        cuv = lsum(accs[4]).astype(jnp.float32)
        x22 = (s1 - s2 - s3 + s4) * 0.25
        x21 = (s2 - s1) * 0.5 - 2.0 * x22
        x12 = (s3 - s1) * 0.5 - 2.0 * x22
        x11 = s1 - 2.0 * x12 - 2.0 * x21 - 4.0 * x22
        a1uv = jnp.where(cuv == 1.0, 1.0, 0.0)
        a2uv = jnp.where(cuv == 2.0, 1.0, 0.0)
        duv = du[0, :]
        dvv = dv[0, :]
        y1u = duv[zi]
        y2u = duv[oi]
        y1v = dvv[zi]
        y2v = dvv[oi]
        c1i = y1u - x11 - x12 - a1uv
        ci1 = y1v - x11 - x21 - a1uv
        c2i = y2u - x21 - x22 - a2uv
        ci2 = y2v - x12 - x22 - a2uv
        tot = 2.0 + x11 + x12 + x21 + c1i + ci1 + x22 + c2i + ci2
        col = jnp.where(lane == 0, 2.0, 0.0)
        for k, ck in ((1, x11), (2, x12), (3, x21), (4, c1i), (5, ci1),
                      (6, x22), (7, c2i), (8, ci2), (9, tot)):
            col = jnp.where(lane == k, ck, col)
        col_v[e, :] = col

    issue(0, bufs_a, sema)

    def pair_body(g, carry):
        e0 = g * 2
        drain(e0, bufs_a, sema)
        issue(e0 + 1, bufs_b, semb)
        compute(e0, bufs_a)
        drain(e0 + 1, bufs_b, semb)

        @pl.when(g < _EPW // 2 - 1)
        def _next():
            issue(e0 + 2, bufs_a, sema)

        compute(e0 + 1, bufs_b)
        return 0

    lax.fori_loop(0, _EPW // 2, pair_body, 0)
    pltpu.sync_copy(col_v, out_hbm.at[pl.ds(base, _EPW), :])


def _sc_counts(u, v, cmat, dtab):
    u16 = jnp.zeros((E, 16), jnp.int32).at[:, 0].set(u).reshape(-1)
    v16 = jnp.zeros((E, 16), jnp.int32).at[:, 0].set(v).reshape(-1)
    mesh = plsc.VectorSubcoreMesh(core_axis_name="c", subcore_axis_name="s")
    f32 = jnp.float32
    rowbuf = pltpu.VMEM((1, N // 2), jnp.int32)
    degbuf = pltpu.VMEM((1, 128), f32)
    scratch = [
        pltpu.VMEM((_EPW * 16,), jnp.int32),
        pltpu.VMEM((_EPW * 16,), jnp.int32),
        rowbuf, rowbuf, degbuf, degbuf,
        rowbuf, rowbuf, degbuf, degbuf,
        pltpu.VMEM((_EPW, 16), f32),
        pltpu.SemaphoreType.DMA,
        pltpu.SemaphoreType.DMA,
    ]
    fn = pl.kernel(
        _sc_counts_body,
        out_type=jax.ShapeDtypeStruct((E, 16), f32),
        mesh=mesh,
        scratch_types=scratch,
    )
    return fn(u16, v16, cmat, dtab)


_BE = 1024

_PAIRS = ((0, 0), (1, 1), (1, 2), (2, 1), (1, 3), (3, 1), (2, 2), (2, 3),
          (3, 2))


def _mlp_body(cnt, z, w1, b1, w2, b2, w3, b3, out):
    zt = z[:]
    t = jnp.stack([zt[a] + zt[b] for a, b in _PAIRS]
                  + [jnp.zeros((H,), jnp.float32)] * 7)
    cblk = cnt[:]
    pooled = jnp.dot(cblk, t, preferred_element_type=jnp.float32,
                     precision=lax.Precision.HIGHEST)
    total = cblk[:, 9:10]
    out_t = pooled / total
    h1 = jnp.maximum(jnp.dot(out_t, w1[:],
                             preferred_element_type=jnp.float32)
                     + b1[:][None, :], 0.0)
    h2 = jnp.maximum(jnp.dot(h1, w2[:],
                             preferred_element_type=jnp.float32)
                     + b2[:][None, :], 0.0)
    logit = jnp.dot(h2, w3[:],
                    preferred_element_type=jnp.float32) + b3[0]
    out[:] = jnp.broadcast_to(logit, (_BE, 128))


def _mlp_head(counts, z_table, W1, b1, W2, b2, W3, b3):
    grid = (E // _BE,)
    logit = pl.pallas_call(
        _mlp_body,
        grid=grid,
        in_specs=[
            pl.BlockSpec((_BE, 16), lambda e: (e, 0)),
            pl.BlockSpec((4, H), lambda e: (0, 0)),
            pl.BlockSpec((H, H), lambda e: (0, 0)),
            pl.BlockSpec((H,), lambda e: (0,)),
            pl.BlockSpec((H, H), lambda e: (0, 0)),
            pl.BlockSpec((H,), lambda e: (0,)),
            pl.BlockSpec((H, 1), lambda e: (0, 0)),
            pl.BlockSpec(memory_space=pltpu.SMEM),
        ],
        out_specs=pl.BlockSpec((_BE, 128), lambda e: (e, 0)),
        out_shape=jax.ShapeDtypeStruct((E, 128), jnp.float32),
    )(counts, z_table, W1, b1, W2, b2, W3, b3)
    return logit[:, :1]


def kernel(x, adj, edges, z_table, W1, b1, W2, b2, W3, b3):
    del x
    a1_bf = adj.astype(jnp.bfloat16)
    cmat, dtab = _compute_a2(a1_bf)
    u = edges[0]
    v = edges[1]
    counts = _sc_counts(u, v, cmat, dtab)
    return _mlp_head(counts, z_table, W1, b1, W2, b2, W3, b3)
